# Initial kernel scaffold; baseline (speedup 1.0000x reference)
import jax
import jax.numpy as jnp
from jax.experimental import pallas as pl


def kernel(x, edge_index, weight, bias):
    raise NotImplementedError("write your pallas kernel here")


# speedup vs baseline: 79.7362x; 79.7362x over previous
import functools

import jax
import jax.numpy as jnp
from jax import lax
from jax.experimental import pallas as pl
from jax.experimental.pallas import tpu as pltpu
from jax.experimental.pallas import tpu_sc as plsc

N = 50000
Q = 4
G = 32
HF = 64
E = 800000

NC = 2
NS = 16
LANES = 16

EPAD = 819200
EROWS = EPAD // 128
NPAD_DEG = 51200
NPAD_ACC = 51200
BATCH = 1024
SUB = 8
KA_BATCHES = EPAD // BATCH // (NC * NS)
KC_BATCHES = EPAD // BATCH // NS
DEG_STRIPE = NPAD_DEG // NS
ACC_STRIPE = NPAD_ACC // NS
ZCHUNK = 64
WAVE = 4
BN = 400


def _mesh():
    return plsc.VectorSubcoreMesh(core_axis_name="c", subcore_axis_name="s",
                                  num_cores=NC, num_subcores=NS)


def _ka_body(rowp, colp, mrow_out, degp, rvm, cvm, mrvm, ones_vm, zbuf, degsh):
    cid = lax.axis_index("c")
    sid = lax.axis_index("s")
    wid = cid * NS + sid

    def fill_z(k, _):
        zbuf[pl.ds(k * 16, 16)] = jnp.zeros((16,), jnp.float32)
        return _
    lax.fori_loop(0, DEG_STRIPE // 16, fill_z, None)
    for i in range(128 // 16):
        ones_vm[pl.ds(i * 16, 16)] = jnp.ones((16,), jnp.float32)

    pltpu.sync_copy(zbuf, degsh.at[pl.ds(sid * DEG_STRIPE, DEG_STRIPE)])
    plsc.subcore_barrier()

    garb = (N + wid * 16) + lax.iota(jnp.int32, 16)

    def batch(b, _):
        rb = (wid * KA_BATCHES + b) * SUB
        pltpu.sync_copy(rowp.at[pl.ds(rb, SUB)], rvm)
        pltpu.sync_copy(colp.at[pl.ds(rb, SUB)], cvm)

        def inner(j, _):
            for i in range(128 // 16):
                sl = pl.ds(i * 16, 16)
                r = rvm[j, sl]
                c = cvm[j, sl]
                mrvm[j, sl] = jnp.where(r == c, garb, r)
            return _
        lax.fori_loop(0, SUB, inner, None)

        for j in range(SUB):
            pltpu.sync_copy(ones_vm, degsh.at[mrvm.at[j]], add=True)
        pltpu.sync_copy(mrvm, mrow_out.at[pl.ds(rb, SUB)])
        return _
    lax.fori_loop(0, KA_BATCHES, batch, None)

    plsc.subcore_barrier()
    sl = pl.ds(sid * DEG_STRIPE, DEG_STRIPE)
    pltpu.sync_copy(degsh.at[sl], degp.at[cid, sl])


def _run_ka(rowp, colp):
    f = pl.kernel(
        _ka_body,
        out_type=[
            jax.ShapeDtypeStruct((EROWS, 128), jnp.int32),
            jax.ShapeDtypeStruct((NC, NPAD_DEG), jnp.float32),
        ],
        mesh=_mesh(),
        compiler_params=pltpu.CompilerParams(use_tc_tiling_on_sc=False),
        scratch_types=[
            pltpu.VMEM((SUB, 128), jnp.int32),
            pltpu.VMEM((SUB, 128), jnp.int32),
            pltpu.VMEM((SUB, 128), jnp.int32),
            pltpu.VMEM((128,), jnp.float32),
            pltpu.VMEM((DEG_STRIPE,), jnp.float32),
            pltpu.VMEM_SHARED((NPAD_DEG,), jnp.float32),
        ],
    )
    return f(rowp, colp)


def _kc_body(mrowp, colp, insf, accum_out, mrvm, cvm, cadj, pay, zpay, sem,
             accsh):
    cid = lax.axis_index("c")
    sid = lax.axis_index("s")

    def fill_z(k, _):
        for i in range(2):
            zpay[k, pl.ds(i * 16, 16)] = jnp.zeros((16,), jnp.float32)
        return _
    lax.fori_loop(0, ZCHUNK, fill_z, None)
    nzero = ACC_STRIPE // ZCHUNK

    for qi in range(NC):
        q = cid * NC + qi
        qbase = q * N

        def zero_stripe(k, _):
            pltpu.sync_copy(
                zpay, accsh.at[pl.ds(sid * ACC_STRIPE + k * ZCHUNK, ZCHUNK)])
            return _
        lax.fori_loop(0, nzero, zero_stripe, None)
        plsc.subcore_barrier()

        def batch(b, _):
            rb = (sid * KC_BATCHES + b) * SUB
            pltpu.sync_copy(mrowp.at[pl.ds(rb, SUB)], mrvm)
            pltpu.sync_copy(colp.at[pl.ds(rb, SUB)], cvm)

            def adj(j, _):
                for i in range(128 // 16):
                    sl = pl.ds(i * 16, 16)
                    cadj[j, sl] = cvm[j, sl] + qbase
                return _
            lax.fori_loop(0, SUB, adj, None)

            for w in range(SUB // WAVE):
                descs = [
                    pltpu.async_copy(insf.at[cadj.at[w * WAVE + j]],
                                     pay.at[pl.ds(j * 128, 128)], sem)
                    for j in range(WAVE)
                ]
                for d in descs:
                    d.wait()
                for j in range(WAVE):
                    pltpu.sync_copy(pay.at[pl.ds(j * 128, 128)],
                                    accsh.at[mrvm.at[w * WAVE + j]], add=True)
            return _
        lax.fori_loop(0, KC_BATCHES, batch, None)

        plsc.subcore_barrier()
        pltpu.sync_copy(
            accsh.at[pl.ds(sid * ACC_STRIPE, ACC_STRIPE)],
            accum_out.at[pl.ds(q * NPAD_ACC + sid * ACC_STRIPE, ACC_STRIPE)])


def _run_kc(mrowp, colp, insf):
    f = pl.kernel(
        _kc_body,
        out_type=jax.ShapeDtypeStruct((Q * NPAD_ACC, G), jnp.float32),
        mesh=_mesh(),
        compiler_params=pltpu.CompilerParams(use_tc_tiling_on_sc=False),
        scratch_types=[
            pltpu.VMEM((SUB, 128), jnp.int32),
            pltpu.VMEM((SUB, 128), jnp.int32),
            pltpu.VMEM((SUB, 128), jnp.int32),
            pltpu.VMEM((WAVE * 128, G), jnp.float32),
            pltpu.VMEM((ZCHUNK, G), jnp.float32),
            pltpu.SemaphoreType.DMA,
            pltpu.VMEM_SHARED((NPAD_ACC, G), jnp.float32),
        ],
    )
    return f(mrowp, colp, insf)


def _kb_body(xb, wc, bb, degb, p_ref, z1_ref, ins1_ref, d_ref):
    deg = degb[:, 0] + degb[:, 1]
    d = jnp.where(deg > 0.0, lax.rsqrt(deg), 0.0)
    z = lax.dot_general(xb[...], wc[...], (((1,), (0,)), ((), ())),
                        preferred_element_type=jnp.float32)
    z0 = z[:, :G]
    z1 = z[:, G:2 * G]
    z2 = z[:, 2 * G:]
    p_ref[...] = z0 - z2 + bb[0, :][None, :]
    z1_ref[...] = z1
    ins1_ref[...] = z2 * d[:, None]
    d_ref[...] = d[:, None]


def _run_kb(xf, wc, bias2, degt):
    nb = Q * N // BN
    return pl.pallas_call(
        _kb_body,
        grid=(nb,),
        in_specs=[
            pl.BlockSpec((BN, HF), lambda i: (i, 0)),
            pl.BlockSpec((HF, 3 * G), lambda i: (0, 0)),
            pl.BlockSpec((1, G), lambda i: (0, 0)),
            pl.BlockSpec((BN, NC), lambda i: (i % (N // BN), 0)),
        ],
        out_specs=[
            pl.BlockSpec((BN, G), lambda i: (i, 0)),
            pl.BlockSpec((BN, G), lambda i: (i, 0)),
            pl.BlockSpec((BN, G), lambda i: (i, 0)),
            pl.BlockSpec((BN, 1), lambda i: (i % (N // BN), 0)),
        ],
        out_shape=[
            jax.ShapeDtypeStruct((Q * N, G), jnp.float32),
            jax.ShapeDtypeStruct((Q * N, G), jnp.float32),
            jax.ShapeDtypeStruct((Q * N, G), jnp.float32),
            jax.ShapeDtypeStruct((N, 1), jnp.float32),
        ],
    )(xf, wc, bias2, degt)


def _kd_body(z1b, accb, db, out_ref):
    d = db[:, 0]
    out_ref[...] = z1b[...] * d[:, None] - (2.0 * d * d)[:, None] * accb[...]


def _acc_spec():
    nq = NPAD_ACC // BN
    nn = N // BN
    return pl.BlockSpec((BN, G), lambda i: ((i // nn) * nq + (i % nn), 0))


def _run_kd(z1f, acc1, dvec):
    nb = Q * N // BN
    return pl.pallas_call(
        _kd_body,
        grid=(nb,),
        in_specs=[
            pl.BlockSpec((BN, G), lambda i: (i, 0)),
            _acc_spec(),
            pl.BlockSpec((BN, 1), lambda i: (i % (N // BN), 0)),
        ],
        out_specs=pl.BlockSpec((BN, G), lambda i: (i, 0)),
        out_shape=jax.ShapeDtypeStruct((Q * N, G), jnp.float32),
    )(z1f, acc1, dvec)


def _ke_body(pb, accb, db, out_ref):
    d = db[:, 0]
    out_ref[...] = pb[...] - d[:, None] * accb[...]


def _run_ke(pf, acc2, dvec):
    nb = Q * N // BN
    return pl.pallas_call(
        _ke_body,
        grid=(nb,),
        in_specs=[
            pl.BlockSpec((BN, G), lambda i: (i, 0)),
            _acc_spec(),
            pl.BlockSpec((BN, 1), lambda i: (i % (N // BN), 0)),
        ],
        out_specs=pl.BlockSpec((BN, G), lambda i: (i, 0)),
        out_shape=jax.ShapeDtypeStruct((Q * N, G), jnp.float32),
    )(pf, acc2, dvec)


@jax.jit
def kernel(x, edge_index, weight, bias):
    ei = jnp.concatenate(
        [edge_index, jnp.zeros((2, EPAD - E), jnp.int32)], axis=1)
    rowp = ei[0].reshape(EROWS, 128)
    colp = ei[1].reshape(EROWS, 128)

    mrowp, degp = _run_ka(rowp, colp)
    degt = jnp.transpose(degp)

    xf = x.reshape(Q * N, HF)
    wc = jnp.transpose(weight, (1, 2, 0, 3)).reshape(HF, 3 * G)
    bias2 = bias.reshape(1, G)
    pf, z1f, ins1f, dvec = _run_kb(xf, wc, bias2, degt)

    acc1 = _run_kc(mrowp, colp, ins1f)
    ins2f = _run_kd(z1f, acc1, dvec)
    acc2 = _run_kc(mrowp, colp, ins2f)
    outf = _run_ke(pf, acc2, dvec)
    return outf.reshape(Q, N, G)


# speedup vs baseline: 96.0590x; 1.2047x over previous
import functools

import jax
import jax.numpy as jnp
from jax import lax
from jax.experimental import pallas as pl
from jax.experimental.pallas import tpu as pltpu
from jax.experimental.pallas import tpu_sc as plsc

N = 50000
Q = 4
G = 32
HF = 64
E = 800000

NC = 2
NS = 16
LANES = 16

EPAD = 819200
EROWS = EPAD // 128
NPAD_DEG = 51200
NPAD_ACC = 51200
NPAD_OUT = 52000
BATCH = 1024
SUB = 8
KA_BATCHES = EPAD // BATCH // (NC * NS)
KC_BATCHES = EPAD // BATCH // NS
DEG_STRIPE = NPAD_DEG // NS
ACC_STRIPE = NPAD_ACC // NS
ZCHUNK = 64
WAVE = 4
BN = 2000


def _mesh():
    return plsc.VectorSubcoreMesh(core_axis_name="c", subcore_axis_name="s",
                                  num_cores=NC, num_subcores=NS)


def _ka_body(rowp, colp, mrow_out, degp, rvm, cvm, mrvm, ones_vm, zbuf, degsh):
    cid = lax.axis_index("c")
    sid = lax.axis_index("s")
    wid = cid * NS + sid

    def fill_z(k, _):
        zbuf[pl.ds(k * 16, 16)] = jnp.zeros((16,), jnp.float32)
        return _
    lax.fori_loop(0, DEG_STRIPE // 16, fill_z, None)
    for i in range(128 // 16):
        ones_vm[pl.ds(i * 16, 16)] = jnp.ones((16,), jnp.float32)

    pltpu.sync_copy(zbuf, degsh.at[pl.ds(sid * DEG_STRIPE, DEG_STRIPE)])
    plsc.subcore_barrier()

    garb = (N + wid * 16) + lax.iota(jnp.int32, 16)

    def batch(b, _):
        rb = (wid * KA_BATCHES + b) * SUB
        pltpu.sync_copy(rowp.at[pl.ds(rb, SUB)], rvm)
        pltpu.sync_copy(colp.at[pl.ds(rb, SUB)], cvm)

        def inner(j, _):
            for i in range(128 // 16):
                sl = pl.ds(i * 16, 16)
                r = rvm[j, sl]
                c = cvm[j, sl]
                mrvm[j, sl] = jnp.where(r == c, garb, r)
            return _
        lax.fori_loop(0, SUB, inner, None)

        for j in range(SUB):
            pltpu.sync_copy(ones_vm, degsh.at[mrvm.at[j]], add=True)
        pltpu.sync_copy(mrvm, mrow_out.at[pl.ds(rb, SUB)])
        return _
    lax.fori_loop(0, KA_BATCHES, batch, None)

    plsc.subcore_barrier()
    sl = pl.ds(sid * DEG_STRIPE, DEG_STRIPE)
    pltpu.sync_copy(degsh.at[sl], degp.at[cid, sl])


def _run_ka(rowp, colp):
    f = pl.kernel(
        _ka_body,
        out_type=[
            jax.ShapeDtypeStruct((EROWS, 128), jnp.int32),
            jax.ShapeDtypeStruct((NC, NPAD_DEG), jnp.float32),
        ],
        mesh=_mesh(),
        compiler_params=pltpu.CompilerParams(use_tc_tiling_on_sc=False),
        scratch_types=[
            pltpu.VMEM((SUB, 128), jnp.int32),
            pltpu.VMEM((SUB, 128), jnp.int32),
            pltpu.VMEM((SUB, 128), jnp.int32),
            pltpu.VMEM((128,), jnp.float32),
            pltpu.VMEM((DEG_STRIPE,), jnp.float32),
            pltpu.VMEM_SHARED((NPAD_DEG,), jnp.float32),
        ],
    )
    return f(rowp, colp)


def _kc_body(mrowp, colp, insf, accum_out, mrvm, cvm, cadj, pay, zpay, sem,
             accsh):
    cid = lax.axis_index("c")
    sid = lax.axis_index("s")

    def fill_z(k, _):
        for i in range(2):
            zpay[k, pl.ds(i * 16, 16)] = jnp.zeros((16,), jnp.float32)
        return _
    lax.fori_loop(0, ZCHUNK, fill_z, None)
    nzero = ACC_STRIPE // ZCHUNK

    for qi in range(NC):
        q = cid * NC + qi
        qbase = q * N

        def zero_stripe(k, _):
            pltpu.sync_copy(
                zpay, accsh.at[pl.ds(sid * ACC_STRIPE + k * ZCHUNK, ZCHUNK)])
            return _
        lax.fori_loop(0, nzero, zero_stripe, None)
        plsc.subcore_barrier()

        def batch(b, _):
            rb = (sid * KC_BATCHES + b) * SUB
            pltpu.sync_copy(mrowp.at[pl.ds(rb, SUB)], mrvm)
            pltpu.sync_copy(colp.at[pl.ds(rb, SUB)], cvm)

            def adj(j, _):
                for i in range(128 // 16):
                    sl = pl.ds(i * 16, 16)
                    cadj[j, sl] = cvm[j, sl] + qbase
                return _
            lax.fori_loop(0, SUB, adj, None)

            for w in range(SUB // WAVE):
                descs = [
                    pltpu.async_copy(insf.at[cadj.at[w * WAVE + j]],
                                     pay.at[pl.ds(j * 128, 128)], sem)
                    for j in range(WAVE)
                ]
                for d in descs:
                    d.wait()
                for j in range(WAVE):
                    pltpu.sync_copy(pay.at[pl.ds(j * 128, 128)],
                                    accsh.at[mrvm.at[w * WAVE + j]], add=True)
            return _
        lax.fori_loop(0, KC_BATCHES, batch, None)

        plsc.subcore_barrier()
        pltpu.sync_copy(
            accsh.at[pl.ds(sid * ACC_STRIPE, ACC_STRIPE)],
            accum_out.at[pl.ds(q * NPAD_OUT + sid * ACC_STRIPE, ACC_STRIPE)])


def _run_kc(mrowp, colp, insf):
    f = pl.kernel(
        _kc_body,
        out_type=jax.ShapeDtypeStruct((Q * NPAD_OUT, G), jnp.float32),
        mesh=_mesh(),
        compiler_params=pltpu.CompilerParams(use_tc_tiling_on_sc=False),
        scratch_types=[
            pltpu.VMEM((SUB, 128), jnp.int32),
            pltpu.VMEM((SUB, 128), jnp.int32),
            pltpu.VMEM((SUB, 128), jnp.int32),
            pltpu.VMEM((WAVE * 128, G), jnp.float32),
            pltpu.VMEM((ZCHUNK, G), jnp.float32),
            pltpu.SemaphoreType.DMA,
            pltpu.VMEM_SHARED((NPAD_ACC, G), jnp.float32),
        ],
    )
    return f(mrowp, colp, insf)


def _kb_body(xb, wc, bb, degb, p_ref, z1_ref, ins1_ref, d_ref):
    deg = degb[:, 0] + degb[:, 1]
    d = jnp.where(deg > 0.0, lax.rsqrt(deg), 0.0)
    z = lax.dot_general(xb[...], wc[...], (((1,), (0,)), ((), ())),
                        preferred_element_type=jnp.float32)
    z0 = z[:, :G]
    z1 = z[:, G:2 * G]
    z2 = z[:, 2 * G:]
    p_ref[...] = z0 - z2 + bb[0, :][None, :]
    z1_ref[...] = z1
    ins1_ref[...] = z2 * d[:, None]
    d_ref[...] = d[:, None]


def _run_kb(xf, wc, bias2, degt):
    nb = Q * N // BN
    return pl.pallas_call(
        _kb_body,
        grid=(nb,),
        in_specs=[
            pl.BlockSpec((BN, HF), lambda i: (i, 0)),
            pl.BlockSpec((HF, 3 * G), lambda i: (0, 0)),
            pl.BlockSpec((1, G), lambda i: (0, 0)),
            pl.BlockSpec((BN, NC), lambda i: (i % (N // BN), 0)),
        ],
        out_specs=[
            pl.BlockSpec((BN, G), lambda i: (i, 0)),
            pl.BlockSpec((BN, G), lambda i: (i, 0)),
            pl.BlockSpec((BN, G), lambda i: (i, 0)),
            pl.BlockSpec((BN, 1), lambda i: (i % (N // BN), 0)),
        ],
        out_shape=[
            jax.ShapeDtypeStruct((Q * N, G), jnp.float32),
            jax.ShapeDtypeStruct((Q * N, G), jnp.float32),
            jax.ShapeDtypeStruct((Q * N, G), jnp.float32),
            jax.ShapeDtypeStruct((N, 1), jnp.float32),
        ],
    )(xf, wc, bias2, degt)


def _kd_body(z1b, accb, db, out_ref):
    d = db[:, 0]
    out_ref[...] = z1b[...] * d[:, None] - (2.0 * d * d)[:, None] * accb[...]


def _acc_spec():
    nq = NPAD_OUT // BN
    nn = N // BN
    return pl.BlockSpec((BN, G), lambda i: ((i // nn) * nq + (i % nn), 0))


def _run_kd(z1f, acc1, dvec):
    nb = Q * N // BN
    return pl.pallas_call(
        _kd_body,
        grid=(nb,),
        in_specs=[
            pl.BlockSpec((BN, G), lambda i: (i, 0)),
            _acc_spec(),
            pl.BlockSpec((BN, 1), lambda i: (i % (N // BN), 0)),
        ],
        out_specs=pl.BlockSpec((BN, G), lambda i: (i, 0)),
        out_shape=jax.ShapeDtypeStruct((Q * N, G), jnp.float32),
    )(z1f, acc1, dvec)


def _ke_body(pb, accb, db, out_ref):
    d = db[:, 0]
    out_ref[...] = pb[...] - d[:, None] * accb[...]


def _run_ke(pf, acc2, dvec):
    nb = Q * N // BN
    return pl.pallas_call(
        _ke_body,
        grid=(nb,),
        in_specs=[
            pl.BlockSpec((BN, G), lambda i: (i, 0)),
            _acc_spec(),
            pl.BlockSpec((BN, 1), lambda i: (i % (N // BN), 0)),
        ],
        out_specs=pl.BlockSpec((BN, G), lambda i: (i, 0)),
        out_shape=jax.ShapeDtypeStruct((Q * N, G), jnp.float32),
    )(pf, acc2, dvec)


@jax.jit
def kernel(x, edge_index, weight, bias):
    ei = jnp.concatenate(
        [edge_index, jnp.zeros((2, EPAD - E), jnp.int32)], axis=1)
    rowp = ei[0].reshape(EROWS, 128)
    colp = ei[1].reshape(EROWS, 128)

    mrowp, degp = _run_ka(rowp, colp)
    degt = jnp.transpose(degp)

    xf = x.reshape(Q * N, HF)
    wc = jnp.transpose(weight, (1, 2, 0, 3)).reshape(HF, 3 * G)
    bias2 = bias.reshape(1, G)
    pf, z1f, ins1f, dvec = _run_kb(xf, wc, bias2, degt)

    acc1 = _run_kc(mrowp, colp, ins1f)
    ins2f = _run_kd(z1f, acc1, dvec)
    acc2 = _run_kc(mrowp, colp, ins2f)
    outf = _run_ke(pf, acc2, dvec)
    return outf.reshape(Q, N, G)


# speedup vs baseline: 118.7422x; 1.2361x over previous
import functools

import jax
import jax.numpy as jnp
from jax import lax
from jax.experimental import pallas as pl
from jax.experimental.pallas import tpu as pltpu
from jax.experimental.pallas import tpu_sc as plsc

N = 50000
Q = 4
G = 32
HF = 64
E = 800000

NC = 2
NS = 16
LANES = 16

EPAD = 819200
EROWS = EPAD // 128
NPAD_DEG = 51200
NPAD_ACC = 51200
NPAD_OUT = 52000
BATCH = 1024
SUB = 8
KA_BATCHES = EPAD // BATCH // (NC * NS)
KC_BATCHES = EPAD // BATCH // NS
DEG_STRIPE = NPAD_DEG // NS
ACC_STRIPE = NPAD_ACC // NS
ZCHUNK = 64
WAVE = 4
BN = 2000


def _mesh():
    return plsc.VectorSubcoreMesh(core_axis_name="c", subcore_axis_name="s",
                                  num_cores=NC, num_subcores=NS)


def _ka_body(rowp, colp, mrow_out, degp, rvm, cvm, mrvm, ones_vm, zbuf, degsh):
    cid = lax.axis_index("c")
    sid = lax.axis_index("s")
    wid = cid * NS + sid

    def fill_z(k, _):
        zbuf[pl.ds(k * 16, 16)] = jnp.zeros((16,), jnp.float32)
        return _
    lax.fori_loop(0, DEG_STRIPE // 16, fill_z, None)
    for i in range(128 // 16):
        ones_vm[pl.ds(i * 16, 16)] = jnp.ones((16,), jnp.float32)

    pltpu.sync_copy(zbuf, degsh.at[pl.ds(sid * DEG_STRIPE, DEG_STRIPE)])
    plsc.subcore_barrier()

    garb = (N + wid * 16) + lax.iota(jnp.int32, 16)

    def batch(b, _):
        rb = (wid * KA_BATCHES + b) * SUB
        pltpu.sync_copy(rowp.at[pl.ds(rb, SUB)], rvm)
        pltpu.sync_copy(colp.at[pl.ds(rb, SUB)], cvm)

        def inner(j, _):
            for i in range(128 // 16):
                sl = pl.ds(i * 16, 16)
                r = rvm[j, sl]
                c = cvm[j, sl]
                mrvm[j, sl] = jnp.where(r == c, garb, r)
            return _
        lax.fori_loop(0, SUB, inner, None)

        for j in range(SUB):
            pltpu.sync_copy(ones_vm, degsh.at[mrvm.at[j]], add=True)
        pltpu.sync_copy(mrvm, mrow_out.at[pl.ds(rb, SUB)])
        return _
    lax.fori_loop(0, KA_BATCHES, batch, None)

    plsc.subcore_barrier()
    sl = pl.ds(sid * DEG_STRIPE, DEG_STRIPE)
    pltpu.sync_copy(degsh.at[sl], degp.at[cid, sl])


def _run_ka(rowp, colp):
    f = pl.kernel(
        _ka_body,
        out_type=[
            jax.ShapeDtypeStruct((EROWS, 128), jnp.int32),
            jax.ShapeDtypeStruct((NC, NPAD_DEG), jnp.float32),
        ],
        mesh=_mesh(),
        compiler_params=pltpu.CompilerParams(use_tc_tiling_on_sc=False),
        scratch_types=[
            pltpu.VMEM((SUB, 128), jnp.int32),
            pltpu.VMEM((SUB, 128), jnp.int32),
            pltpu.VMEM((SUB, 128), jnp.int32),
            pltpu.VMEM((128,), jnp.float32),
            pltpu.VMEM((DEG_STRIPE,), jnp.float32),
            pltpu.VMEM_SHARED((NPAD_DEG,), jnp.float32),
        ],
    )
    return f(rowp, colp)


NB = KC_BATCHES
NU = NB * SUB
RING = 5
IB = 3


def _kc_body(mrowp, colp, insf, accum_out, mrvm, cvm, pay, sem_i, sem_g,
             sem_s, accsh):
    cid = lax.axis_index("c")
    sid = lax.axis_index("s")

    def wait_idx():
        pltpu.make_async_copy(mrowp.at[pl.ds(0, SUB)], mrvm.at[0],
                              sem_i).wait()

    def wait_pay(sem):
        pltpu.make_async_copy(insf.at[pl.ds(0, 128)], pay.at[0], sem).wait()

    for qi in range(NC):
        q = cid * NC + qi
        qbase = q * N

        def fill_z(k, _):
            for i in range(2):
                pay[0, k, pl.ds(i * 16, 16)] = jnp.zeros((16,), jnp.float32)
            return _
        lax.fori_loop(0, 128, fill_z, None)

        def zero_stripe(k, _):
            pltpu.sync_copy(
                pay.at[0], accsh.at[pl.ds(sid * ACC_STRIPE + k * 128, 128)])
            return _
        lax.fori_loop(0, ACC_STRIPE // 128, zero_stripe, None)
        plsc.subcore_barrier()

        rb0 = sid * NB * SUB
        pltpu.async_copy(mrowp.at[pl.ds(rb0, SUB)], mrvm.at[0], sem_i)
        pltpu.async_copy(colp.at[pl.ds(rb0, SUB)], cvm.at[0], sem_i)

        def unit(u, _):
            b = u // SUB
            j = u - b * SUB
            hb = lax.rem(b, IB)
            rbuf = lax.rem(u, RING)

            @pl.when(jnp.logical_and(u < NU, j == 0))
            def _preamble():
                wait_idx()
                wait_idx()

                def adj(j2, _):
                    for i in range(8):
                        sl = pl.ds(i * 16, 16)
                        cvm[hb, j2, sl] = cvm[hb, j2, sl] + qbase
                    return _
                lax.fori_loop(0, SUB, adj, None)

                @pl.when(b + 1 < NB)
                def _prefetch():
                    rb = (sid * NB + b + 1) * SUB
                    nhb = lax.rem(b + 1, IB)
                    pltpu.async_copy(mrowp.at[pl.ds(rb, SUB)], mrvm.at[nhb],
                                     sem_i)
                    pltpu.async_copy(colp.at[pl.ds(rb, SUB)], cvm.at[nhb],
                                     sem_i)

            @pl.when(jnp.logical_and(u >= RING, u < NU))
            def _free_ring():
                wait_pay(sem_s)

            @pl.when(u < NU)
            def _gather():
                pltpu.async_copy(insf.at[cvm.at[hb, j]], pay.at[rbuf], sem_g)

            @pl.when(u >= 2)
            def _scatter():
                v = u - 2
                bv = v // SUB
                jv = v - bv * SUB
                hv = lax.rem(bv, IB)
                rv = lax.rem(v, RING)
                wait_pay(sem_g)
                pltpu.async_copy(pay.at[rv], accsh.at[mrvm.at[hv, jv]],
                                 sem_s, add=True)
            return _
        lax.fori_loop(0, NU + 2, unit, None)

        for _ in range(RING):
            wait_pay(sem_s)

        plsc.subcore_barrier()
        pltpu.sync_copy(
            accsh.at[pl.ds(sid * ACC_STRIPE, ACC_STRIPE)],
            accum_out.at[pl.ds(q * NPAD_OUT + sid * ACC_STRIPE, ACC_STRIPE)])


def _run_kc(mrowp, colp, insf):
    f = pl.kernel(
        _kc_body,
        out_type=jax.ShapeDtypeStruct((Q * NPAD_OUT, G), jnp.float32),
        mesh=_mesh(),
        compiler_params=pltpu.CompilerParams(use_tc_tiling_on_sc=False),
        scratch_types=[
            pltpu.VMEM((IB, SUB, 128), jnp.int32),
            pltpu.VMEM((IB, SUB, 128), jnp.int32),
            pltpu.VMEM((RING, 128, G), jnp.float32),
            pltpu.SemaphoreType.DMA,
            pltpu.SemaphoreType.DMA,
            pltpu.SemaphoreType.DMA,
            pltpu.VMEM_SHARED((NPAD_ACC, G), jnp.float32),
        ],
    )
    return f(mrowp, colp, insf)


def _kb_body(xb, wc, bb, degb, p_ref, z1_ref, ins1_ref, d_ref):
    deg = degb[:, 0] + degb[:, 1]
    d = jnp.where(deg > 0.0, lax.rsqrt(deg), 0.0)
    z = lax.dot_general(xb[...], wc[...], (((1,), (0,)), ((), ())),
                        preferred_element_type=jnp.float32)
    z0 = z[:, :G]
    z1 = z[:, G:2 * G]
    z2 = z[:, 2 * G:]
    p_ref[...] = z0 - z2 + bb[0, :][None, :]
    z1_ref[...] = z1
    ins1_ref[...] = z2 * d[:, None]
    d_ref[...] = d[:, None]


def _run_kb(xf, wc, bias2, degt):
    nb = Q * N // BN
    return pl.pallas_call(
        _kb_body,
        grid=(nb,),
        in_specs=[
            pl.BlockSpec((BN, HF), lambda i: (i, 0)),
            pl.BlockSpec((HF, 3 * G), lambda i: (0, 0)),
            pl.BlockSpec((1, G), lambda i: (0, 0)),
            pl.BlockSpec((BN, NC), lambda i: (i % (N // BN), 0)),
        ],
        out_specs=[
            pl.BlockSpec((BN, G), lambda i: (i, 0)),
            pl.BlockSpec((BN, G), lambda i: (i, 0)),
            pl.BlockSpec((BN, G), lambda i: (i, 0)),
            pl.BlockSpec((BN, 1), lambda i: (i % (N // BN), 0)),
        ],
        out_shape=[
            jax.ShapeDtypeStruct((Q * N, G), jnp.float32),
            jax.ShapeDtypeStruct((Q * N, G), jnp.float32),
            jax.ShapeDtypeStruct((Q * N, G), jnp.float32),
            jax.ShapeDtypeStruct((N, 1), jnp.float32),
        ],
    )(xf, wc, bias2, degt)


def _kd_body(z1b, accb, db, out_ref):
    d = db[:, 0]
    out_ref[...] = z1b[...] * d[:, None] - (2.0 * d * d)[:, None] * accb[...]


def _acc_spec():
    nq = NPAD_OUT // BN
    nn = N // BN
    return pl.BlockSpec((BN, G), lambda i: ((i // nn) * nq + (i % nn), 0))


def _run_kd(z1f, acc1, dvec):
    nb = Q * N // BN
    return pl.pallas_call(
        _kd_body,
        grid=(nb,),
        in_specs=[
            pl.BlockSpec((BN, G), lambda i: (i, 0)),
            _acc_spec(),
            pl.BlockSpec((BN, 1), lambda i: (i % (N // BN), 0)),
        ],
        out_specs=pl.BlockSpec((BN, G), lambda i: (i, 0)),
        out_shape=jax.ShapeDtypeStruct((Q * N, G), jnp.float32),
    )(z1f, acc1, dvec)


def _ke_body(pb, accb, db, out_ref):
    d = db[:, 0]
    out_ref[...] = pb[...] - d[:, None] * accb[...]


def _run_ke(pf, acc2, dvec):
    nb = Q * N // BN
    return pl.pallas_call(
        _ke_body,
        grid=(nb,),
        in_specs=[
            pl.BlockSpec((BN, G), lambda i: (i, 0)),
            _acc_spec(),
            pl.BlockSpec((BN, 1), lambda i: (i % (N // BN), 0)),
        ],
        out_specs=pl.BlockSpec((BN, G), lambda i: (i, 0)),
        out_shape=jax.ShapeDtypeStruct((Q * N, G), jnp.float32),
    )(pf, acc2, dvec)


@jax.jit
def kernel(x, edge_index, weight, bias):
    ei = jnp.concatenate(
        [edge_index, jnp.zeros((2, EPAD - E), jnp.int32)], axis=1)
    rowp = ei[0].reshape(EROWS, 128)
    colp = ei[1].reshape(EROWS, 128)

    mrowp, degp = _run_ka(rowp, colp)
    degt = jnp.transpose(degp)

    xf = x.reshape(Q * N, HF)
    wc = jnp.transpose(weight, (1, 2, 0, 3)).reshape(HF, 3 * G)
    bias2 = bias.reshape(1, G)
    pf, z1f, ins1f, dvec = _run_kb(xf, wc, bias2, degt)

    acc1 = _run_kc(mrowp, colp, ins1f)
    ins2f = _run_kd(z1f, acc1, dvec)
    acc2 = _run_kc(mrowp, colp, ins2f)
    outf = _run_ke(pf, acc2, dvec)
    return outf.reshape(Q, N, G)


# speedup vs baseline: 119.5700x; 1.0070x over previous
import functools

import jax
import jax.numpy as jnp
from jax import lax
from jax.experimental import pallas as pl
from jax.experimental.pallas import tpu as pltpu
from jax.experimental.pallas import tpu_sc as plsc

N = 50000
Q = 4
G = 32
HF = 64
E = 800000

NC = 2
NS = 16
LANES = 16

EPAD = 819200
EROWS = EPAD // 128
NPAD_DEG = 51200
NPAD_ACC = 51200
NPAD_OUT = 52000
BATCH = 1024
SUB = 8
KA_BATCHES = EPAD // BATCH // (NC * NS)
KC_BATCHES = EPAD // BATCH // NS
DEG_STRIPE = NPAD_DEG // NS
ACC_STRIPE = NPAD_ACC // NS
ZCHUNK = 64
WAVE = 4
BN = 2000


def _mesh():
    return plsc.VectorSubcoreMesh(core_axis_name="c", subcore_axis_name="s",
                                  num_cores=NC, num_subcores=NS)


def _ka_body(rowp, colp, mrow_out, degp, rvm, cvm, mrvm, ones_vm, zbuf, degsh):
    cid = lax.axis_index("c")
    sid = lax.axis_index("s")
    wid = cid * NS + sid

    def fill_z(k, _):
        zbuf[pl.ds(k * 16, 16)] = jnp.zeros((16,), jnp.float32)
        return _
    lax.fori_loop(0, DEG_STRIPE // 16, fill_z, None)
    for i in range(128 // 16):
        ones_vm[pl.ds(i * 16, 16)] = jnp.ones((16,), jnp.float32)

    pltpu.sync_copy(zbuf, degsh.at[pl.ds(sid * DEG_STRIPE, DEG_STRIPE)])
    plsc.subcore_barrier()

    garb = (N + wid * 16) + lax.iota(jnp.int32, 16)

    def batch(b, _):
        rb = (wid * KA_BATCHES + b) * SUB
        pltpu.sync_copy(rowp.at[pl.ds(rb, SUB)], rvm)
        pltpu.sync_copy(colp.at[pl.ds(rb, SUB)], cvm)

        def inner(j, _):
            for i in range(128 // 16):
                sl = pl.ds(i * 16, 16)
                r = rvm[j, sl]
                c = cvm[j, sl]
                mrvm[j, sl] = jnp.where(r == c, garb, r)
            return _
        lax.fori_loop(0, SUB, inner, None)

        for j in range(SUB):
            pltpu.sync_copy(ones_vm, degsh.at[mrvm.at[j]], add=True)
        pltpu.sync_copy(mrvm, mrow_out.at[pl.ds(rb, SUB)])
        return _
    lax.fori_loop(0, KA_BATCHES, batch, None)

    plsc.subcore_barrier()
    sl = pl.ds(sid * DEG_STRIPE, DEG_STRIPE)
    pltpu.sync_copy(degsh.at[sl], degp.at[cid, sl])


def _run_ka(rowp, colp):
    f = pl.kernel(
        _ka_body,
        out_type=[
            jax.ShapeDtypeStruct((EROWS, 128), jnp.int32),
            jax.ShapeDtypeStruct((NC, NPAD_DEG), jnp.float32),
        ],
        mesh=_mesh(),
        compiler_params=pltpu.CompilerParams(use_tc_tiling_on_sc=False),
        scratch_types=[
            pltpu.VMEM((SUB, 128), jnp.int32),
            pltpu.VMEM((SUB, 128), jnp.int32),
            pltpu.VMEM((SUB, 128), jnp.int32),
            pltpu.VMEM((128,), jnp.float32),
            pltpu.VMEM((DEG_STRIPE,), jnp.float32),
            pltpu.VMEM_SHARED((NPAD_DEG,), jnp.float32),
        ],
    )
    return f(rowp, colp)


NB = KC_BATCHES
NU = NB * SUB
RING = 5
IB = 3


def _kc_body(mrowp, colp, insf, accum_out, mrvm, cvm, pay, sem_i, sem_g,
             sem_s, accsh):
    cid = lax.axis_index("c")
    sid = lax.axis_index("s")

    def wait_idx():
        pltpu.make_async_copy(mrowp.at[pl.ds(0, SUB)], mrvm.at[0],
                              sem_i).wait()

    def wait_pay(sem):
        pltpu.make_async_copy(insf.at[pl.ds(0, 128)], pay.at[0], sem).wait()

    for qi in range(NC):
        q = cid * NC + qi
        qbase = q * N

        def fill_z(k, _):
            for i in range(2):
                pay[0, k, pl.ds(i * 16, 16)] = jnp.zeros((16,), jnp.float32)
            return _
        lax.fori_loop(0, 128, fill_z, None)

        def zero_stripe(k, _):
            pltpu.sync_copy(
                pay.at[0], accsh.at[pl.ds(sid * ACC_STRIPE + k * 128, 128)])
            return _
        lax.fori_loop(0, ACC_STRIPE // 128, zero_stripe, None)
        plsc.subcore_barrier()

        rb0 = sid * NB * SUB
        pltpu.async_copy(mrowp.at[pl.ds(rb0, SUB)], mrvm.at[0], sem_i)
        pltpu.async_copy(colp.at[pl.ds(rb0, SUB)], cvm.at[0], sem_i)

        def unit(u, _):
            b = u // SUB
            j = u - b * SUB
            hb = lax.rem(b, IB)
            rbuf = lax.rem(u, RING)

            @pl.when(jnp.logical_and(u < NU, j == 0))
            def _preamble():
                wait_idx()
                wait_idx()

                def adj(j2, _):
                    for i in range(8):
                        sl = pl.ds(i * 16, 16)
                        cvm[hb, j2, sl] = cvm[hb, j2, sl] + qbase
                    return _
                lax.fori_loop(0, SUB, adj, None)

                @pl.when(b + 1 < NB)
                def _prefetch():
                    rb = (sid * NB + b + 1) * SUB
                    nhb = lax.rem(b + 1, IB)
                    pltpu.async_copy(mrowp.at[pl.ds(rb, SUB)], mrvm.at[nhb],
                                     sem_i)
                    pltpu.async_copy(colp.at[pl.ds(rb, SUB)], cvm.at[nhb],
                                     sem_i)

            @pl.when(jnp.logical_and(u >= RING, u < NU))
            def _free_ring():
                wait_pay(sem_s)

            @pl.when(u < NU)
            def _gather():
                pltpu.async_copy(insf.at[cvm.at[hb, j]], pay.at[rbuf], sem_g)

            @pl.when(u >= 3)
            def _scatter():
                v = u - 3
                bv = v // SUB
                jv = v - bv * SUB
                hv = lax.rem(bv, IB)
                rv = lax.rem(v, RING)
                wait_pay(sem_g)
                pltpu.async_copy(pay.at[rv], accsh.at[mrvm.at[hv, jv]],
                                 sem_s, add=True)
            return _
        lax.fori_loop(0, NU + 3, unit, None)

        for _ in range(RING):
            wait_pay(sem_s)

        plsc.subcore_barrier()
        pltpu.sync_copy(
            accsh.at[pl.ds(sid * ACC_STRIPE, ACC_STRIPE)],
            accum_out.at[pl.ds(q * NPAD_OUT + sid * ACC_STRIPE, ACC_STRIPE)])


def _run_kc(mrowp, colp, insf):
    f = pl.kernel(
        _kc_body,
        out_type=jax.ShapeDtypeStruct((Q * NPAD_OUT, G), jnp.float32),
        mesh=_mesh(),
        compiler_params=pltpu.CompilerParams(use_tc_tiling_on_sc=False),
        scratch_types=[
            pltpu.VMEM((IB, SUB, 128), jnp.int32),
            pltpu.VMEM((IB, SUB, 128), jnp.int32),
            pltpu.VMEM((RING, 128, G), jnp.float32),
            pltpu.SemaphoreType.DMA,
            pltpu.SemaphoreType.DMA,
            pltpu.SemaphoreType.DMA,
            pltpu.VMEM_SHARED((NPAD_ACC, G), jnp.float32),
        ],
    )
    return f(mrowp, colp, insf)


def _kb_body(xb, wc, bb, degb, p_ref, z1_ref, ins1_ref, d_ref):
    deg = degb[:, 0] + degb[:, 1]
    d = jnp.where(deg > 0.0, lax.rsqrt(deg), 0.0)
    z = lax.dot_general(xb[...], wc[...], (((1,), (0,)), ((), ())),
                        preferred_element_type=jnp.float32)
    z0 = z[:, :G]
    z1 = z[:, G:2 * G]
    z2 = z[:, 2 * G:]
    p_ref[...] = z0 - z2 + bb[0, :][None, :]
    z1_ref[...] = z1
    ins1_ref[...] = z2 * d[:, None]
    d_ref[...] = d[:, None]


def _run_kb(xf, wc, bias2, degt):
    nb = Q * N // BN
    return pl.pallas_call(
        _kb_body,
        grid=(nb,),
        in_specs=[
            pl.BlockSpec((BN, HF), lambda i: (i, 0)),
            pl.BlockSpec((HF, 3 * G), lambda i: (0, 0)),
            pl.BlockSpec((1, G), lambda i: (0, 0)),
            pl.BlockSpec((BN, NC), lambda i: (i % (N // BN), 0)),
        ],
        out_specs=[
            pl.BlockSpec((BN, G), lambda i: (i, 0)),
            pl.BlockSpec((BN, G), lambda i: (i, 0)),
            pl.BlockSpec((BN, G), lambda i: (i, 0)),
            pl.BlockSpec((BN, 1), lambda i: (i % (N // BN), 0)),
        ],
        out_shape=[
            jax.ShapeDtypeStruct((Q * N, G), jnp.float32),
            jax.ShapeDtypeStruct((Q * N, G), jnp.float32),
            jax.ShapeDtypeStruct((Q * N, G), jnp.float32),
            jax.ShapeDtypeStruct((N, 1), jnp.float32),
        ],
    )(xf, wc, bias2, degt)


def _kd_body(z1b, accb, db, out_ref):
    d = db[:, 0]
    out_ref[...] = z1b[...] * d[:, None] - (2.0 * d * d)[:, None] * accb[...]


def _acc_spec():
    nq = NPAD_OUT // BN
    nn = N // BN
    return pl.BlockSpec((BN, G), lambda i: ((i // nn) * nq + (i % nn), 0))


def _run_kd(z1f, acc1, dvec):
    nb = Q * N // BN
    return pl.pallas_call(
        _kd_body,
        grid=(nb,),
        in_specs=[
            pl.BlockSpec((BN, G), lambda i: (i, 0)),
            _acc_spec(),
            pl.BlockSpec((BN, 1), lambda i: (i % (N // BN), 0)),
        ],
        out_specs=pl.BlockSpec((BN, G), lambda i: (i, 0)),
        out_shape=jax.ShapeDtypeStruct((Q * N, G), jnp.float32),
    )(z1f, acc1, dvec)


def _ke_body(pb, accb, db, out_ref):
    d = db[:, 0]
    out_ref[...] = pb[...] - d[:, None] * accb[...]


def _run_ke(pf, acc2, dvec):
    nb = Q * N // BN
    return pl.pallas_call(
        _ke_body,
        grid=(nb,),
        in_specs=[
            pl.BlockSpec((BN, G), lambda i: (i, 0)),
            _acc_spec(),
            pl.BlockSpec((BN, 1), lambda i: (i % (N // BN), 0)),
        ],
        out_specs=pl.BlockSpec((BN, G), lambda i: (i, 0)),
        out_shape=jax.ShapeDtypeStruct((Q * N, G), jnp.float32),
    )(pf, acc2, dvec)


@jax.jit
def kernel(x, edge_index, weight, bias):
    ei = jnp.concatenate(
        [edge_index, jnp.zeros((2, EPAD - E), jnp.int32)], axis=1)
    rowp = ei[0].reshape(EROWS, 128)
    colp = ei[1].reshape(EROWS, 128)

    mrowp, degp = _run_ka(rowp, colp)
    degt = jnp.transpose(degp)

    xf = x.reshape(Q * N, HF)
    wc = jnp.transpose(weight, (1, 2, 0, 3)).reshape(HF, 3 * G)
    bias2 = bias.reshape(1, G)
    pf, z1f, ins1f, dvec = _run_kb(xf, wc, bias2, degt)

    acc1 = _run_kc(mrowp, colp, ins1f)
    ins2f = _run_kd(z1f, acc1, dvec)
    acc2 = _run_kc(mrowp, colp, ins2f)
    outf = _run_ke(pf, acc2, dvec)
    return outf.reshape(Q, N, G)


# speedup vs baseline: 190.3472x; 1.5919x over previous
import functools

import jax
import jax.numpy as jnp
from jax import lax
from jax.experimental import pallas as pl
from jax.experimental.pallas import tpu as pltpu
from jax.experimental.pallas import tpu_sc as plsc

N = 50000
Q = 4
G = 32
HF = 64
E = 800000

NC = 2
NS = 16
LANES = 16

EPAD = 819200
EROWS = EPAD // 128
NPAD_DEG = 51200
NPAD_ACC = 51200
NPAD_OUT = 52000
BATCH = 1024
SUB = 8
KA_BATCHES = EPAD // BATCH // (NC * NS)
KC_BATCHES = EPAD // BATCH // NS
DEG_STRIPE = NPAD_DEG // NS
ACC_STRIPE = NPAD_ACC // NS
ZCHUNK = 64
WAVE = 4
BN = 2000


def _mesh():
    return plsc.VectorSubcoreMesh(core_axis_name="c", subcore_axis_name="s",
                                  num_cores=NC, num_subcores=NS)


def _ka_body(rowp, colp, mrow_out, degp, rvm, cvm, mrvm, ones_vm, zbuf, degsh):
    cid = lax.axis_index("c")
    sid = lax.axis_index("s")
    wid = cid * NS + sid

    def fill_z(k, _):
        zbuf[pl.ds(k * 16, 16)] = jnp.zeros((16,), jnp.float32)
        return _
    lax.fori_loop(0, DEG_STRIPE // 16, fill_z, None)
    for i in range(128 // 16):
        ones_vm[pl.ds(i * 16, 16)] = jnp.ones((16,), jnp.float32)

    pltpu.sync_copy(zbuf, degsh.at[pl.ds(sid * DEG_STRIPE, DEG_STRIPE)])
    plsc.subcore_barrier()

    garb = (N + wid * 16) + lax.iota(jnp.int32, 16)

    def batch(b, _):
        rb = (wid * KA_BATCHES + b) * SUB
        pltpu.sync_copy(rowp.at[pl.ds(rb, SUB)], rvm)
        pltpu.sync_copy(colp.at[pl.ds(rb, SUB)], cvm)

        def inner(j, _):
            for i in range(128 // 16):
                sl = pl.ds(i * 16, 16)
                r = rvm[j, sl]
                c = cvm[j, sl]
                mrvm[j, sl] = jnp.where(r == c, garb, r)
            return _
        lax.fori_loop(0, SUB, inner, None)

        for j in range(SUB):
            pltpu.sync_copy(ones_vm, degsh.at[mrvm.at[j]], add=True)
        pltpu.sync_copy(mrvm, mrow_out.at[pl.ds(rb, SUB)])
        return _
    lax.fori_loop(0, KA_BATCHES, batch, None)

    plsc.subcore_barrier()
    sl = pl.ds(sid * DEG_STRIPE, DEG_STRIPE)
    pltpu.sync_copy(degsh.at[sl], degp.at[cid, sl])


def _run_ka(rowp, colp):
    f = pl.kernel(
        _ka_body,
        out_type=[
            jax.ShapeDtypeStruct((EROWS, 128), jnp.int32),
            jax.ShapeDtypeStruct((NC, NPAD_DEG), jnp.float32),
        ],
        mesh=_mesh(),
        compiler_params=pltpu.CompilerParams(use_tc_tiling_on_sc=False),
        scratch_types=[
            pltpu.VMEM((SUB, 128), jnp.int32),
            pltpu.VMEM((SUB, 128), jnp.int32),
            pltpu.VMEM((SUB, 128), jnp.int32),
            pltpu.VMEM((128,), jnp.float32),
            pltpu.VMEM((DEG_STRIPE,), jnp.float32),
            pltpu.VMEM_SHARED((NPAD_DEG,), jnp.float32),
        ],
    )
    return f(rowp, colp)


NB = KC_BATCHES
NU = NB * SUB
RING = 5
IB = 3


def _kc_body(mrowp, colp, insf, accum_out, mrvm, cvm, pay, sem_i, sem_g,
             sem_s, accsh):
    cid = lax.axis_index("c")
    sid = lax.axis_index("s")

    def wait_idx():
        pltpu.make_async_copy(mrowp.at[pl.ds(0, SUB)], mrvm.at[0],
                              sem_i).wait()

    def wait_pay(sem):
        pltpu.make_async_copy(insf.at[pl.ds(0, 128)], pay.at[0], sem).wait()

    for qi in range(NC):
        q = cid * NC + qi
        qbase = q * N

        def fill_z(k, _):
            for i in range(2):
                pay[0, k, pl.ds(i * 16, 16)] = jnp.zeros((16,), jnp.float32)
            return _
        lax.fori_loop(0, 128, fill_z, None)

        def zero_stripe(k, _):
            pltpu.sync_copy(
                pay.at[0], accsh.at[pl.ds(sid * ACC_STRIPE + k * 128, 128)])
            return _
        lax.fori_loop(0, ACC_STRIPE // 128, zero_stripe, None)
        plsc.subcore_barrier()

        rb0 = sid * NB * SUB
        pltpu.async_copy(mrowp.at[pl.ds(rb0, SUB)], mrvm.at[0], sem_i)
        pltpu.async_copy(colp.at[pl.ds(rb0, SUB)], cvm.at[0], sem_i)

        def unit(u, _):
            b = u // SUB
            j = u - b * SUB
            hb = lax.rem(b, IB)
            rbuf = lax.rem(u, RING)

            @pl.when(jnp.logical_and(u < NU, j == 0))
            def _preamble():
                wait_idx()
                wait_idx()

                def adj(j2, _):
                    for i in range(8):
                        sl = pl.ds(i * 16, 16)
                        cvm[hb, j2, sl] = cvm[hb, j2, sl] + qbase
                    return _
                lax.fori_loop(0, SUB, adj, None)

                @pl.when(b + 1 < NB)
                def _prefetch():
                    rb = (sid * NB + b + 1) * SUB
                    nhb = lax.rem(b + 1, IB)
                    pltpu.async_copy(mrowp.at[pl.ds(rb, SUB)], mrvm.at[nhb],
                                     sem_i)
                    pltpu.async_copy(colp.at[pl.ds(rb, SUB)], cvm.at[nhb],
                                     sem_i)

            @pl.when(jnp.logical_and(u >= RING, u < NU))
            def _free_ring():
                wait_pay(sem_s)

            @pl.when(u < NU)
            def _gather():
                pltpu.async_copy(insf.at[cvm.at[hb, j]], pay.at[rbuf], sem_g)

            @pl.when(u >= 3)
            def _scatter():
                v = u - 3
                bv = v // SUB
                jv = v - bv * SUB
                hv = lax.rem(bv, IB)
                rv = lax.rem(v, RING)
                wait_pay(sem_g)
                pltpu.async_copy(pay.at[rv], accsh.at[mrvm.at[hv, jv]],
                                 sem_s, add=True)
            return _
        lax.fori_loop(0, NU + 3, unit, None)

        for _ in range(RING):
            wait_pay(sem_s)

        plsc.subcore_barrier()
        pltpu.sync_copy(
            accsh.at[pl.ds(sid * ACC_STRIPE, ACC_STRIPE)],
            accum_out.at[pl.ds(q * NPAD_OUT + sid * ACC_STRIPE, ACC_STRIPE)])


def _run_kc(mrowp, colp, insf):
    f = pl.kernel(
        _kc_body,
        out_type=jax.ShapeDtypeStruct((Q * NPAD_OUT, G), jnp.float32),
        mesh=_mesh(),
        compiler_params=pltpu.CompilerParams(use_tc_tiling_on_sc=False),
        scratch_types=[
            pltpu.VMEM((IB, SUB, 128), jnp.int32),
            pltpu.VMEM((IB, SUB, 128), jnp.int32),
            pltpu.VMEM((RING, 128, G), jnp.float32),
            pltpu.SemaphoreType.DMA,
            pltpu.SemaphoreType.DMA,
            pltpu.SemaphoreType.DMA,
            pltpu.VMEM_SHARED((NPAD_ACC, G), jnp.float32),
        ],
    )
    return f(mrowp, colp, insf)


def _kb_body(xb, wc, bb, degb, p_ref, z1_ref, ins1_ref, d_ref):
    deg = degb[:, 0] + degb[:, 1]
    d = jnp.where(deg > 0.0, lax.rsqrt(deg), 0.0)
    z = lax.dot_general(xb[...], wc[...], (((1,), (0,)), ((), ())),
                        preferred_element_type=jnp.float32)
    z0 = z[:, :G]
    z1 = z[:, G:2 * G]
    z2 = z[:, 2 * G:]
    p_ref[...] = z0 - z2 + bb[0, :][None, :]
    z1_ref[...] = z1
    ins1_ref[...] = z2 * d[:, None]
    d_ref[...] = d[:, None]


def _run_kb(xf, wc, bias2, degt):
    nb = Q * N // BN
    return pl.pallas_call(
        _kb_body,
        grid=(nb,),
        in_specs=[
            pl.BlockSpec((BN, HF), lambda i: (i, 0)),
            pl.BlockSpec((HF, 3 * G), lambda i: (0, 0)),
            pl.BlockSpec((1, G), lambda i: (0, 0)),
            pl.BlockSpec((BN, NC), lambda i: (i % (N // BN), 0)),
        ],
        out_specs=[
            pl.BlockSpec((BN, G), lambda i: (i, 0)),
            pl.BlockSpec((BN, G), lambda i: (i, 0)),
            pl.BlockSpec((BN, G), lambda i: (i, 0)),
            pl.BlockSpec((BN, 1), lambda i: (i % (N // BN), 0)),
        ],
        out_shape=[
            jax.ShapeDtypeStruct((Q * N, G), jnp.float32),
            jax.ShapeDtypeStruct((Q * N, G), jnp.float32),
            jax.ShapeDtypeStruct((Q * N, G), jnp.float32),
            jax.ShapeDtypeStruct((N, 1), jnp.float32),
        ],
    )(xf, wc, bias2, degt)


def _kd_body(z1b, accb, db, out_ref):
    d = db[:, 0]
    out_ref[...] = z1b[...] * d[:, None] - (2.0 * d * d)[:, None] * accb[...]


def _acc_spec():
    nq = NPAD_OUT // BN
    nn = N // BN
    return pl.BlockSpec((BN, G), lambda i: ((i // nn) * nq + (i % nn), 0))


def _run_kd(z1f, acc1, dvec):
    nb = Q * N // BN
    return pl.pallas_call(
        _kd_body,
        grid=(nb,),
        in_specs=[
            pl.BlockSpec((BN, G), lambda i: (i, 0)),
            _acc_spec(),
            pl.BlockSpec((BN, 1), lambda i: (i % (N // BN), 0)),
        ],
        out_specs=pl.BlockSpec((BN, G), lambda i: (i, 0)),
        out_shape=jax.ShapeDtypeStruct((Q * N, G), jnp.float32),
    )(z1f, acc1, dvec)


def _ke_body(pb, accb, db, out_ref):
    d = db[:, 0]
    out_ref[...] = pb[...] - d[:, None] * accb[...]


def _run_ke(pf, acc2, dvec):
    nb = Q * N // BN
    return pl.pallas_call(
        _ke_body,
        grid=(nb,),
        in_specs=[
            pl.BlockSpec((BN, G), lambda i: (i, 0)),
            _acc_spec(),
            pl.BlockSpec((BN, 1), lambda i: (i % (N // BN), 0)),
        ],
        out_specs=pl.BlockSpec((BN, G), lambda i: (i, 0)),
        out_shape=jax.ShapeDtypeStruct((Q * N, G), jnp.float32),
    )(pf, acc2, dvec)


@jax.jit
def kernel(x, edge_index, weight, bias):
    pad = jnp.arange(EPAD - E, dtype=jnp.int32) % N
    ei = jnp.concatenate(
        [edge_index, jnp.stack([pad, pad])], axis=1)
    rowp = ei[0].reshape(EROWS, 128)
    colp = ei[1].reshape(EROWS, 128)

    mrowp, degp = _run_ka(rowp, colp)
    degt = jnp.transpose(degp)

    xf = x.reshape(Q * N, HF)
    wc = jnp.transpose(weight, (1, 2, 0, 3)).reshape(HF, 3 * G)
    bias2 = bias.reshape(1, G)
    pf, z1f, ins1f, dvec = _run_kb(xf, wc, bias2, degt)

    acc1 = _run_kc(mrowp, colp, ins1f)
    ins2f = _run_kd(z1f, acc1, dvec)
    acc2 = _run_kc(mrowp, colp, ins2f)
    outf = _run_ke(pf, acc2, dvec)
    return outf.reshape(Q, N, G)
